# Initial kernel scaffold; baseline (speedup 1.0000x reference)
import jax
import jax.numpy as jnp
from jax.experimental import pallas as pl


def kernel(x, edge_index, batch, W_te, b_te, W_in, b_in, Wc, bc, ln_g, ln_b, W_out, b_out, W_ih, W_hh, b_ih, b_hh, W1, b1, W2, b2, W3, b3):
    raise NotImplementedError("write your pallas kernel here")


# speedup vs baseline: 3.6407x; 3.6407x over previous
import functools

import jax
import jax.numpy as jnp
from jax import lax
from jax.experimental import pallas as pl
from jax.experimental.pallas import tpu as pltpu
from jax.experimental.pallas import tpu_sc as plsc

N = 10000
E = 320000
F = 128
H = 128
OUT = 4
T = 3
K = 3
L = 3
B = 16
FS = 1

NC = 2
NS = 16
NW = NC * NS
EPT = E // NW
EBR = 125
EB = 128
EPB = EPT // EBR
IHALF = 40
NP = 10240
RPT = NP // NS
DEGW = 128

_f32 = jnp.float32


def _sc_mesh():
    return plsc.VectorSubcoreMesh(core_axis_name="c", subcore_axis_name="s",
                                  num_cores=NC, num_subcores=NS)


def _segsum_body(z_hbm, src1_hbm, dst1_hbm, zeros_hbm, out_hbm,
                 srcv0, srcv1, dstv0, dstv1, rows, acc, gsem0, gsem1):
    c = lax.axis_index("c")
    s = lax.axis_index("s")
    wid = c * NS + s
    ebase = pl.multiple_of(wid * EPB * EB, 8)
    rowbase = pl.multiple_of(s * RPT, 8)
    srcvs = (srcv0, srcv1)
    dstvs = (dstv0, dstv1)
    sems = (gsem0, gsem1)
    pltpu.async_copy(zeros_hbm, acc.at[pl.ds(rowbase, RPT)], gsem0).wait()
    plsc.subcore_barrier()

    pltpu.sync_copy(src1_hbm.at[pl.ds(ebase, EB)], srcv0)
    pltpu.sync_copy(dst1_hbm.at[pl.ds(ebase, EB)], dstv0)
    descs = [pltpu.async_copy(z_hbm.at[srcv0], rows.at[0], sems[0])]
    for b in range(EPB):
        p = b % 2
        q = (b + 1) % 2
        if b + 1 < EPB:
            nbase = ebase + (b + 1) * EB
            pltpu.sync_copy(src1_hbm.at[pl.ds(nbase, EB)], srcvs[q])
            pltpu.sync_copy(dst1_hbm.at[pl.ds(nbase, EB)], dstvs[q])
            descs.append(
                pltpu.async_copy(z_hbm.at[srcvs[q]], rows.at[q], sems[q]))
        descs[b].wait()
        pltpu.async_copy(rows.at[p], acc.at[dstvs[p]], sems[p],
                         add=True).wait()

    plsc.subcore_barrier()
    pltpu.async_copy(acc.at[pl.ds(rowbase, RPT)],
                     out_hbm.at[c, pl.ds(rowbase, RPT)], gsem0).wait()


@functools.cache
def _make_sc_segsum():
    return pl.kernel(
        _segsum_body,
        out_type=jax.ShapeDtypeStruct((NC, NP, H), _f32),
        mesh=_sc_mesh(),
        scratch_types=[
            pltpu.VMEM((EB,), jnp.int32),
            pltpu.VMEM((EB,), jnp.int32),
            pltpu.VMEM((EB,), jnp.int32),
            pltpu.VMEM((EB,), jnp.int32),
            pltpu.VMEM((2, EB, H), _f32),
            pltpu.VMEM_SHARED((NP, H), _f32),
            pltpu.SemaphoreType.DMA,
            pltpu.SemaphoreType.DMA,
        ],
        name="sc_segsum",
    )


def _sc_segsum(z, src1, dst1, zeros):
    return _make_sc_segsum()(z, src1, dst1, zeros)


def _degree_body(src1_hbm, ones_hbm, zeros_hbm, out_hbm,
                 srcv0, srcv1, onesv, table, dsem, dsem1):
    c = lax.axis_index("c")
    s = lax.axis_index("s")
    wid = c * NS + s
    ebase = pl.multiple_of(wid * EPB * EB, 8)
    pltpu.sync_copy(ones_hbm, onesv)
    rowbase = pl.multiple_of(s * RPT, 8)
    pltpu.async_copy(zeros_hbm, table.at[pl.ds(rowbase, RPT)], dsem).wait()
    plsc.subcore_barrier()

    srcvs = (srcv0, srcv1)
    sems = (dsem, dsem1)
    pltpu.sync_copy(src1_hbm.at[pl.ds(ebase, EB)], srcv0)
    for b in range(EPB):
        p = b % 2
        if b + 1 < EPB:
            pltpu.sync_copy(src1_hbm.at[pl.ds(ebase + (b + 1) * EB, EB)],
                            srcvs[(b + 1) % 2])
        pltpu.async_copy(onesv, table.at[srcvs[p]], sems[p], add=True).wait()

    plsc.subcore_barrier()
    pltpu.async_copy(table.at[pl.ds(rowbase, RPT)],
                     out_hbm.at[c, pl.ds(rowbase, RPT)], dsem).wait()


@functools.cache
def _make_sc_degree():
    return pl.kernel(
        _degree_body,
        out_type=jax.ShapeDtypeStruct((NC, NP, DEGW), _f32),
        mesh=_sc_mesh(),
        scratch_types=[
            pltpu.VMEM((EB,), jnp.int32),
            pltpu.VMEM((EB,), jnp.int32),
            pltpu.VMEM((EB, DEGW), _f32),
            pltpu.VMEM_SHARED((NP, DEGW), _f32),
            pltpu.SemaphoreType.DMA,
            pltpu.SemaphoreType.DMA,
        ],
        name="sc_degree",
    )


def _sc_degree(src1, ones, zeros):
    return _make_sc_degree()(src1, ones, zeros)


BLK = 1000
GRID = N // BLK


def _dis_blk(degp):
    deg = degp[0, :, 0:1] + degp[1, :, 0:1]
    return jnp.where(deg > 0.0,
                     1.0 / jnp.sqrt(jnp.maximum(deg, 1e-12)),
                     0.0)


def _row_spec():
    return pl.BlockSpec((BLK, H), lambda i: (i, 0))


def _part_spec():
    return pl.BlockSpec((NC, BLK, H), lambda i: (0, i, 0))


def _deg_spec():
    return pl.BlockSpec((NC, BLK, DEGW), lambda i: (0, i, 0))


def _w_spec(r, c):
    return pl.BlockSpec((r, c), lambda i: (0, 0))


def _stage_a_kernel(tfrac, x_ref, win_ref, bin_ref, wte_ref, bte_ref,
                    degp_ref, h_ref, a_ref):
    tf = tfrac * wte_ref[...] + bte_ref[...]
    ct = jnp.dot(tf, win_ref[F:, :], preferred_element_type=_f32) + bin_ref[...]
    h = jnp.dot(x_ref[...], win_ref[:F, :], preferred_element_type=_f32) + ct
    h_ref[...] = h
    a_ref[...] = _dis_blk(degp_ref[...]) * h


def _tc_stage_a(tfrac, x_t, W_in, b_in, W_te, b_te, degp):
    return pl.pallas_call(
        functools.partial(_stage_a_kernel, tfrac),
        grid=(GRID,),
        in_specs=[
            _row_spec(),
            _w_spec(F + H, H),
            _w_spec(1, H),
            _w_spec(1, H),
            _w_spec(1, H),
            _deg_spec(),
        ],
        out_specs=[_row_spec(), _row_spec()],
        out_shape=[jax.ShapeDtypeStruct((N, H), _f32)] * 2,
    )(x_t, W_in, b_in, W_te, b_te, degp)


def _mid_kernel(h_ref, sp_ref, degp_ref, w0_ref, w1_ref, acc_ref, a2_ref):
    dis = _dis_blk(degp_ref[...])
    s1 = sp_ref[0] + sp_ref[1]
    tx1 = -dis * s1
    acc_ref[...] = (jnp.dot(h_ref[...], w0_ref[...], preferred_element_type=_f32)
                    + jnp.dot(tx1, w1_ref[...], preferred_element_type=_f32))
    a2_ref[...] = dis * tx1


def _tc_mid(h, s1p, degp, W0, W1):
    return pl.pallas_call(
        _mid_kernel,
        grid=(GRID,),
        in_specs=[
            _row_spec(),
            _part_spec(),
            _deg_spec(),
            _w_spec(H, H),
            _w_spec(H, H),
        ],
        out_specs=[_row_spec(), _row_spec()],
        out_shape=[jax.ShapeDtypeStruct((N, H), _f32)] * 2,
    )(h, s1p, degp, W0, W1)


def _end_kernel(acc_ref, sp_ref, degp_ref, h_ref, w2_ref, b_ref, g_ref,
                bl_ref, hn_ref, an_ref):
    dis = _dis_blk(degp_ref[...])
    s2 = sp_ref[0] + sp_ref[1]
    h = h_ref[...]
    tx2 = -2.0 * dis * s2 - h
    out = (acc_ref[...]
           + jnp.dot(tx2, w2_ref[...], preferred_element_type=_f32)
           + b_ref[...])
    r = jnp.maximum(out, 0.0) + h
    mu = jnp.mean(r, axis=-1, keepdims=True)
    var = jnp.mean((r - mu) ** 2, axis=-1, keepdims=True)
    hn = (r - mu) / jnp.sqrt(var + 1e-5) * g_ref[...] + bl_ref[...]
    hn_ref[...] = hn
    an_ref[...] = dis * hn


def _tc_end(acc, s2p, degp, h, W2, bci, gi, bli):
    return pl.pallas_call(
        _end_kernel,
        grid=(GRID,),
        in_specs=[
            _row_spec(),
            _part_spec(),
            _deg_spec(),
            _row_spec(),
            _w_spec(H, H),
            _w_spec(1, H),
            _w_spec(1, H),
            _w_spec(1, H),
        ],
        out_specs=[_row_spec(), _row_spec()],
        out_shape=[jax.ShapeDtypeStruct((N, H), _f32)] * 2,
    )(acc, s2p, degp, h, W2, bci, gi, bli)


def _pool_kernel(h_ref, batch_ref, wout_ref, bout_ref, p_ref, c_ref):
    @pl.when(pl.program_id(0) == 0)
    def _():
        p_ref[...] = jnp.zeros_like(p_ref)
        c_ref[...] = jnp.zeros_like(c_ref)

    nf = jnp.dot(h_ref[...], wout_ref[...], preferred_element_type=_f32) + bout_ref[...]
    bvec = batch_ref[0, 0, :]
    onehot = (lax.broadcasted_iota(jnp.int32, (B, BLK), 0)
              == bvec[None, :]).astype(_f32)
    p_ref[...] += jnp.dot(onehot, nf, preferred_element_type=_f32)
    c_ref[...] += jnp.broadcast_to(
        jnp.sum(onehot, axis=1, keepdims=True), (B, H))


def _tc_pool(h, batch3d, W_out, b_out):
    return pl.pallas_call(
        _pool_kernel,
        grid=(GRID,),
        in_specs=[
            _row_spec(),
            pl.BlockSpec((1, 1, BLK), lambda i: (i, 0, 0)),
            _w_spec(H, H),
            _w_spec(1, H),
        ],
        out_specs=[
            pl.BlockSpec((B, H), lambda i: (0, 0)),
            pl.BlockSpec((B, H), lambda i: (0, 0)),
        ],
        out_shape=[jax.ShapeDtypeStruct((B, H), _f32)] * 2,
    )(h, batch3d, W_out, b_out)


def _head_kernel(p_ref, c_ref, wih_ref, whh_ref, bih_ref, bhh_ref,
                 w1_ref, b1_ref, w2_ref, b2_ref, w3_ref, b3_ref, o_ref):
    counts = jnp.maximum(c_ref[...], 1.0)
    h = jnp.zeros((B, H), dtype=_f32)
    c = jnp.zeros((B, H), dtype=_f32)
    for t in range(T):
        ge = p_ref[t] / counts
        g = (jnp.dot(ge, wih_ref[...], preferred_element_type=_f32)
             + bih_ref[...]
             + jnp.dot(h, whh_ref[...], preferred_element_type=_f32)
             + bhh_ref[...])
        i_g = jax.nn.sigmoid(g[:, :H])
        f_g = jax.nn.sigmoid(g[:, H:2 * H])
        g_g = jnp.tanh(g[:, 2 * H:3 * H])
        o_g = jax.nn.sigmoid(g[:, 3 * H:])
        c = f_g * c + i_g * g_g
        h = o_g * jnp.tanh(c)
    m = jnp.maximum(jnp.dot(h, w1_ref[...], preferred_element_type=_f32)
                    + b1_ref[...], 0.0)
    m = jnp.maximum(jnp.dot(m, w2_ref[...], preferred_element_type=_f32)
                    + b2_ref[...], 0.0)
    o_ref[...] = jnp.dot(m, w3_ref[...], preferred_element_type=_f32) + b3_ref[...]


def _tc_head(pooled, counts, W_ihT, W_hhT, b_ih, b_hh, W1, b1, W2, b2, W3, b3):
    return pl.pallas_call(
        _head_kernel,
        out_shape=jax.ShapeDtypeStruct((B, OUT * FS), _f32),
    )(pooled, counts, W_ihT, W_hhT, b_ih, b_hh, W1, b1, W2, b2, W3, b3)


def kernel(x, edge_index, batch, W_te, b_te, W_in, b_in, Wc, bc, ln_g, ln_b,
           W_out, b_out, W_ih, W_hh, b_ih, b_hh, W1, b1, W2, b2, W3, b3):
    nrows = NW * EPB
    src1 = jnp.concatenate(
        [edge_index[0].reshape(nrows, EBR),
         jnp.zeros((nrows, EB - EBR), jnp.int32)], axis=1).reshape(-1)
    junk = jnp.full((nrows, EB - EBR), N, jnp.int32)
    srcd1 = jnp.concatenate(
        [edge_index[0].reshape(nrows, EBR), junk], axis=1).reshape(-1)
    dst1 = jnp.concatenate(
        [edge_index[1].reshape(nrows, EBR), junk], axis=1).reshape(-1)
    zrows = jnp.zeros((RPT, H), dtype=_f32)
    onesd = jnp.ones((EB, DEGW), dtype=_f32)
    zrowsd = jnp.zeros((RPT, DEGW), dtype=_f32)
    batch3d = batch.reshape(GRID, 1, BLK)

    degp = _sc_degree(srcd1, onesd, zrowsd)

    b_te2 = b_te.reshape(1, H)
    b_in2 = b_in.reshape(1, H)

    pooled_ts = []
    counts = None
    for t in range(T):
        h, a = _tc_stage_a(float(t) / T, x[t], W_in, b_in2, W_te, b_te2, degp)
        for i in range(L):
            s1p = _sc_segsum(a, src1, dst1, zrows)
            acc, a2 = _tc_mid(h, s1p, degp, Wc[i, 0], Wc[i, 1])
            s2p = _sc_segsum(a2, src1, dst1, zrows)
            h, a = _tc_end(acc, s2p, degp, h, Wc[i, 2],
                           bc[i].reshape(1, H), ln_g[i].reshape(1, H),
                           ln_b[i].reshape(1, H))
        pooled_t, counts = _tc_pool(h, batch3d, W_out, b_out.reshape(1, H))
        pooled_ts.append(pooled_t)

    pooled = jnp.stack(pooled_ts, axis=0)
    pred = _tc_head(pooled, counts, W_ih.T, W_hh.T,
                    b_ih.reshape(1, 4 * H), b_hh.reshape(1, 4 * H),
                    W1, b1.reshape(1, 2 * H), W2, b2.reshape(1, H),
                    W3, b3.reshape(1, OUT * FS))
    return pred.reshape(B, FS, OUT)
